# baseline (speedup 1.0000x reference)
import jax
import jax.numpy as jnp
from jax.experimental import pallas as pl
from jax.experimental.pallas import tpu as pltpu

_NBUF = 7
_CHUNK = 512
_NSUB = 8
_SUB = _CHUNK // _NSUB


def _router_body(x_hbm, w_ref, b_ref, w_out_ref, i_out_ref, bufs, sems):
    nchunks = x_hbm.shape[0] // _CHUNK
    num_experts = w_ref.shape[0]

    def _subcopies(c, slot):
        return [
            pltpu.make_async_copy(
                x_hbm.at[pl.ds(c * _CHUNK + k * _SUB, _SUB), :],
                bufs.at[slot, pl.ds(k * _SUB, _SUB), :],
                sems.at[slot],
            )
            for k in range(_NSUB)
        ]

    def _start(c, slot):
        for k, cp in enumerate(_subcopies(c, slot)):
            cp.start(priority=k % 2)

    def _wait(c, slot):
        for cp in _subcopies(c, slot):
            cp.wait()

    for c in range(min(_NBUF, nchunks)):
        _start(c, c)

    w = w_ref[...]
    bias = b_ref[...].reshape(num_experts, 1)

    def _step(c, carry):
        slot = jax.lax.rem(c, _NBUF)
        _wait(c, slot)
        logits = jax.lax.dot_general(
            w, bufs[slot],
            dimension_numbers=(((1,), (1,)), ((), ())),
            preferred_element_type=jnp.float32,
        ) + bias
        m = jnp.max(logits, axis=0, keepdims=True)
        s = jnp.sum(jnp.exp(logits - m), axis=0)
        w_out_ref[c, :] = 1.0 / s
        iota = jax.lax.broadcasted_iota(jnp.int32, logits.shape, 0)
        i_out_ref[c, :] = jnp.min(
            jnp.where(logits == m, iota, num_experts), axis=0)

        @pl.when(c + _NBUF < nchunks)
        def _prefetch():
            _start(c + _NBUF, slot)

        return carry

    jax.lax.fori_loop(0, nchunks, _step, 0)


def kernel(x, W, b):
    tokens, d_model = x.shape
    num_experts = W.shape[0]
    nchunks = tokens // _CHUNK
    b2 = b.reshape(1, num_experts)
    weights, indices = pl.pallas_call(
        _router_body,
        in_specs=[
            pl.BlockSpec(memory_space=pl.ANY),
            pl.BlockSpec(memory_space=pltpu.VMEM),
            pl.BlockSpec(memory_space=pltpu.VMEM),
        ],
        out_specs=[
            pl.BlockSpec(memory_space=pltpu.VMEM),
            pl.BlockSpec(memory_space=pltpu.VMEM),
        ],
        out_shape=[
            jax.ShapeDtypeStruct((nchunks, _CHUNK), jnp.float32),
            jax.ShapeDtypeStruct((nchunks, _CHUNK), jnp.int32),
        ],
        scratch_shapes=[
            pltpu.VMEM((_NBUF, _CHUNK, d_model), jnp.float32),
            pltpu.SemaphoreType.DMA((_NBUF,)),
        ],
        compiler_params=pltpu.CompilerParams(
            vmem_limit_bytes=63 * 1024 * 1024,
        ),
    )(x, W, b2)
    return weights.reshape(tokens), indices.reshape(tokens)


# speedup vs baseline: 1.0174x; 1.0174x over previous
import jax
import jax.numpy as jnp
from jax.experimental import pallas as pl
from jax.experimental.pallas import tpu as pltpu

_NBUF = 3
_CHUNK = 1024
_NSUB = 8
_SUB = _CHUNK // _NSUB


def _router_body(x_hbm, w_ref, b_ref, w_out_ref, i_out_ref, bufs, sems):
    nchunks = x_hbm.shape[0] // _CHUNK
    num_experts = w_ref.shape[0]

    def _subcopies(c, slot):
        return [
            pltpu.make_async_copy(
                x_hbm.at[pl.ds(c * _CHUNK + k * _SUB, _SUB), :],
                bufs.at[slot, pl.ds(k * _SUB, _SUB), :],
                sems.at[slot],
            )
            for k in range(_NSUB)
        ]

    def _start(c, slot):
        for k, cp in enumerate(_subcopies(c, slot)):
            cp.start(priority=k % 2)

    def _wait(c, slot):
        for cp in _subcopies(c, slot):
            cp.wait()

    for c in range(min(_NBUF, nchunks)):
        _start(c, c)

    w = w_ref[...]
    bias = b_ref[...].reshape(num_experts, 1)

    def _step(c, carry):
        slot = jax.lax.rem(c, _NBUF)
        _wait(c, slot)
        logits = jax.lax.dot_general(
            w, bufs[slot],
            dimension_numbers=(((1,), (1,)), ((), ())),
            preferred_element_type=jnp.float32,
        ) + bias
        m = jnp.max(logits, axis=0, keepdims=True)
        s = jnp.sum(jnp.exp(logits - m), axis=0)
        w_out_ref[c, :] = 1.0 / s
        iota = jax.lax.broadcasted_iota(jnp.int32, logits.shape, 0)
        i_out_ref[c, :] = jnp.min(
            jnp.where(logits == m, iota, num_experts), axis=0)

        @pl.when(c + _NBUF < nchunks)
        def _prefetch():
            _start(c + _NBUF, slot)

        return carry

    jax.lax.fori_loop(0, nchunks, _step, 0)


def kernel(x, W, b):
    tokens, d_model = x.shape
    num_experts = W.shape[0]
    nchunks = tokens // _CHUNK
    b2 = b.reshape(1, num_experts)
    weights, indices = pl.pallas_call(
        _router_body,
        in_specs=[
            pl.BlockSpec(memory_space=pl.ANY),
            pl.BlockSpec(memory_space=pltpu.VMEM),
            pl.BlockSpec(memory_space=pltpu.VMEM),
        ],
        out_specs=[
            pl.BlockSpec(memory_space=pltpu.VMEM),
            pl.BlockSpec(memory_space=pltpu.VMEM),
        ],
        out_shape=[
            jax.ShapeDtypeStruct((nchunks, _CHUNK), jnp.float32),
            jax.ShapeDtypeStruct((nchunks, _CHUNK), jnp.int32),
        ],
        scratch_shapes=[
            pltpu.VMEM((_NBUF, _CHUNK, d_model), jnp.float32),
            pltpu.SemaphoreType.DMA((_NBUF,)),
        ],
        compiler_params=pltpu.CompilerParams(
            vmem_limit_bytes=63 * 1024 * 1024,
        ),
    )(x, W, b2)
    return weights.reshape(tokens), indices.reshape(tokens)
